# baseline (speedup 1.0000x reference)
import functools

import jax
import jax.numpy as jnp
from jax import lax
from jax.experimental import pallas as pl
from jax.experimental.pallas import tpu as pltpu
from jax.experimental.pallas import tpu_sc as plsc

N = 10000
D = 128
E = 320000
NS = 16
CH = 128
NBUF = 4
ABUF = 2
CPS = 160
CPA = NS * CPS
TOT = 2 * CPA
EPAD = CPA * CH
NDUMP = 64
NP = N + NDUMP
RPT = (N // NS) // 8 * 8
RTL = N - NS * RPT
BLK = 1000


def _deg_body(dst_ref, zeros_ref, ones_ref, out_ref,
              d0_, d1_, d2_, d3_, ones_v, i0, i1, i2, i3, acc_s):
    didx = (d0_, d1_, d2_, d3_)
    isems = (i0, i1, i2, i3)
    c = lax.axis_index("c")
    s = lax.axis_index("s")
    pltpu.sync_copy(ones_ref, ones_v)
    pltpu.sync_copy(zeros_ref.at[pl.ds(s * RPT, RPT)],
                    acc_s.at[pl.ds(s * RPT, RPT)])

    @pl.when(s == NS - 1)
    def _():
        pltpu.sync_copy(zeros_ref.at[pl.ds(NS * RPT, RTL)],
                        acc_s.at[pl.ds(NS * RPT, RTL)])

    plsc.subcore_barrier()
    ebase = c * EPAD + s * CPS * CH

    def wait_idx(b):
        pltpu.make_async_copy(dst_ref.at[pl.ds(0, CH)], didx[b],
                              isems[b]).wait()

    for b in range(4):
        pltpu.async_copy(dst_ref.at[pl.ds(ebase + b * CH, CH)],
                         didx[b], isems[b])

    @pl.loop(0, CPS // 4)
    def _(t):
        for b in range(4):
            wait_idx(b)
            pltpu.sync_copy(ones_v, acc_s.at[didx[b]], add=True)
            pltpu.async_copy(
                dst_ref.at[pl.ds(ebase + (4 * t + b + 4) * CH, CH)],
                didx[b], isems[b])

    for b in range(4):
        wait_idx(b)
    plsc.subcore_barrier()
    pltpu.sync_copy(acc_s.at[pl.ds(s * RPT, RPT)],
                    out_ref.at[c, pl.ds(s * RPT, RPT)])

    @pl.when(s == NS - 1)
    def _():
        pltpu.sync_copy(acc_s.at[pl.ds(NS * RPT, RTL)],
                        out_ref.at[c, pl.ds(NS * RPT, RTL)])


def _agg_body(y_ref, srcf_ref, dstf_ref, out_ref,
              s0_, s1_, s2_, s3_, d0_, d1_, d2_, d3_, r0, r1,
              i0, i1, i2, i3, g0, g1, acc_s):
    sidx = (s0_, s1_, s2_, s3_)
    didx = (d0_, d1_, d2_, d3_)
    rows = (r0, r1)
    isems = (i0, i1, i2, i3)
    gsems = (g0, g1)
    c = lax.axis_index("c")
    s = lax.axis_index("s")
    pltpu.sync_copy(y_ref.at[pl.ds(c * N + s * RPT, RPT)],
                    acc_s.at[pl.ds(s * RPT, RPT)])

    @pl.when(s == NS - 1)
    def _():
        pltpu.sync_copy(y_ref.at[pl.ds(c * N + NS * RPT, RTL)],
                        acc_s.at[pl.ds(NS * RPT, RTL)])

    plsc.subcore_barrier()
    ebase = c * EPAD + s * CPS * CH

    def start_idx(b, j):
        pltpu.async_copy(srcf_ref.at[pl.ds(ebase + j * CH, CH)],
                         sidx[b], isems[b])
        pltpu.async_copy(dstf_ref.at[pl.ds(ebase + j * CH, CH)],
                         didx[b], isems[b])

    def wait_idx(b):
        pltpu.make_async_copy(srcf_ref.at[pl.ds(0, CH)], sidx[b],
                              isems[b]).wait()
        pltpu.make_async_copy(srcf_ref.at[pl.ds(0, CH)], didx[b],
                              isems[b]).wait()

    def wait_gather(rb):
        pltpu.make_async_copy(y_ref.at[pl.ds(0, CH)], rows[rb],
                              gsems[rb]).wait()

    @pl.loop(0, CPS)
    def _(k):
        pltpu.sync_copy(srcf_ref.at[pl.ds(ebase + k * CH, CH)], sidx[0])
        pltpu.sync_copy(dstf_ref.at[pl.ds(ebase + k * CH, CH)], didx[0])
        pltpu.async_copy(y_ref.at[sidx[0]], rows[0], gsems[0]).wait()
        pltpu.sync_copy(rows[0], acc_s.at[didx[0]], add=True)

    plsc.subcore_barrier()
    pltpu.sync_copy(acc_s.at[pl.ds(s * RPT, RPT)],
                    out_ref.at[c, pl.ds(s * RPT, RPT)])

    @pl.when(s == NS - 1)
    def _():
        pltpu.sync_copy(acc_s.at[pl.ds(NS * RPT, RTL)],
                        out_ref.at[c, pl.ds(NS * RPT, RTL)])


def _scale_body(x_ref, w_ref, deg_ref, y_ref):
    xw = jnp.dot(x_ref[...], w_ref[...], preferred_element_type=jnp.float32)
    d0 = lax.rsqrt(deg_ref[0, :, 0:1] + 1.0)
    d1 = lax.rsqrt(deg_ref[1, :, 0:1] + 1.0)
    y_ref[0] = d0 * xw
    y_ref[1] = d1 * xw


_scale_call = pl.pallas_call(
    _scale_body,
    grid=(N // BLK,),
    in_specs=[
        pl.BlockSpec((BLK, D), lambda i: (i, 0)),
        pl.BlockSpec((D, D), lambda i: (0, 0)),
        pl.BlockSpec((2, BLK, D), lambda i: (0, i, 0)),
    ],
    out_specs=pl.BlockSpec((2, BLK, D), lambda i: (0, i, 0)),
    out_shape=jax.ShapeDtypeStruct((2, N, D), jnp.float32),
)


def _final_body(agg_ref, deg_ref, b_ref, out_ref):
    bv = b_ref[0]
    d0 = lax.rsqrt(deg_ref[0, :, 0:1] + 1.0)
    d1 = lax.rsqrt(deg_ref[1, :, 0:1] + 1.0)
    out_ref[:, :D] = d0 * agg_ref[0] + bv
    out_ref[:, D:] = d1 * agg_ref[1] + bv


_final_call = pl.pallas_call(
    _final_body,
    grid=(N // BLK,),
    in_specs=[
        pl.BlockSpec((2, BLK, D), lambda i: (0, i, 0)),
        pl.BlockSpec((2, BLK, D), lambda i: (0, i, 0)),
        pl.BlockSpec((1, D), lambda i: (0, 0)),
    ],
    out_specs=pl.BlockSpec((BLK, 2 * D), lambda i: (i, 0)),
    out_shape=jax.ShapeDtypeStruct((N, 2 * D), jnp.float32),
)


@functools.cache
def _sc_kernels():
    mesh = plsc.VectorSubcoreMesh(core_axis_name="c", subcore_axis_name="s")
    deg_kernel = pl.kernel(
        _deg_body,
        mesh=mesh,
        out_type=jax.ShapeDtypeStruct((2, N, D), jnp.float32),
        scratch_types=[
            pltpu.VMEM((CH,), jnp.int32),
            pltpu.VMEM((CH,), jnp.int32),
            pltpu.VMEM((CH,), jnp.int32),
            pltpu.VMEM((CH,), jnp.int32),
            pltpu.VMEM((CH, D), jnp.float32),
            pltpu.SemaphoreType.DMA,
            pltpu.SemaphoreType.DMA,
            pltpu.SemaphoreType.DMA,
            pltpu.SemaphoreType.DMA,
            pltpu.VMEM_SHARED((NP, D), jnp.float32),
        ],
    )
    agg_kernel = pl.kernel(
        _agg_body,
        mesh=mesh,
        out_type=jax.ShapeDtypeStruct((2, N, D), jnp.float32),
        scratch_types=[
            pltpu.VMEM((CH,), jnp.int32),
            pltpu.VMEM((CH,), jnp.int32),
            pltpu.VMEM((CH,), jnp.int32),
            pltpu.VMEM((CH,), jnp.int32),
            pltpu.VMEM((CH,), jnp.int32),
            pltpu.VMEM((CH,), jnp.int32),
            pltpu.VMEM((CH,), jnp.int32),
            pltpu.VMEM((CH,), jnp.int32),
            pltpu.VMEM((CH, D), jnp.float32),
            pltpu.VMEM((CH, D), jnp.float32),
            pltpu.SemaphoreType.DMA,
            pltpu.SemaphoreType.DMA,
            pltpu.SemaphoreType.DMA,
            pltpu.SemaphoreType.DMA,
            pltpu.SemaphoreType.DMA,
            pltpu.SemaphoreType.DMA,
            pltpu.VMEM_SHARED((NP, D), jnp.float32),
        ],
    )
    return deg_kernel, agg_kernel


def kernel(x, edge_index_list, W, b):
    deg_kernel, agg_kernel = _sc_kernels()
    ei = edge_index_list.astype(jnp.int32)
    src = ei[:, 0, :]
    dst = ei[:, 1, :]
    srcoff = src + jnp.arange(2, dtype=jnp.int32)[:, None] * N
    pad_src = jnp.broadcast_to(jnp.array([[0], [N]], jnp.int32), (2, EPAD - E))
    srcoff_p = jnp.concatenate([srcoff, pad_src], axis=1).reshape(2 * EPAD)
    pad_dst = N + jnp.arange(EPAD - E, dtype=jnp.int32) % NDUMP
    dst_p = jnp.concatenate(
        [dst, jnp.broadcast_to(pad_dst, (2, EPAD - E))], axis=1)
    dst_p = dst_p.reshape(2 * EPAD)
    over = jnp.zeros(8 * CH, jnp.int32)
    srcf = jnp.concatenate([srcoff_p, over])
    dstf = jnp.concatenate([dst_p, over])
    zerosd = jnp.zeros((N, D), jnp.float32)
    onesd = jnp.ones((CH, D), jnp.float32)

    degp = deg_kernel(dstf, zerosd, onesd)
    y = _scale_call(x, W, degp)
    agg = agg_kernel(y.reshape(2 * N, D), srcf, dstf)
    return _final_call(agg, degp, b.reshape(1, D))


# speedup vs baseline: 1.4317x; 1.4317x over previous
import functools

import jax
import jax.numpy as jnp
from jax import lax
from jax.experimental import pallas as pl
from jax.experimental.pallas import tpu as pltpu
from jax.experimental.pallas import tpu_sc as plsc

N = 10000
D = 128
E = 320000
NS = 16
CH = 128
ET = E // NS
NCH = ET // CH
TAIL = ET - NCH * CH
RPT = (N // NS) // 8 * 8
RTL = N - NS * RPT
BLK = 1000


def _deg_body(dst_ref, zeros_ref, ones_ref, out_ref,
              idx_v, idx_t, ones_v, acc_s, sem):
    c = lax.axis_index("c")
    s = lax.axis_index("s")
    pltpu.sync_copy(ones_ref, ones_v)
    pltpu.sync_copy(zeros_ref.at[pl.ds(s * RPT, RPT)],
                    acc_s.at[pl.ds(s * RPT, RPT)])

    @pl.when(s == NS - 1)
    def _():
        pltpu.sync_copy(zeros_ref.at[pl.ds(NS * RPT, RTL)],
                        acc_s.at[pl.ds(NS * RPT, RTL)])

    plsc.subcore_barrier()
    base = c * E + s * ET

    @pl.loop(0, NCH)
    def _(k):
        pltpu.sync_copy(dst_ref.at[pl.ds(base + k * CH, CH)], idx_v)
        pltpu.sync_copy(ones_v, acc_s.at[idx_v], add=True)

    pltpu.sync_copy(dst_ref.at[pl.ds(base + NCH * CH, TAIL)], idx_t)
    pltpu.sync_copy(ones_v.at[pl.ds(0, TAIL)], acc_s.at[idx_t], add=True)
    plsc.subcore_barrier()
    pltpu.sync_copy(acc_s.at[pl.ds(s * RPT, RPT)],
                    out_ref.at[c, pl.ds(s * RPT, RPT)])

    @pl.when(s == NS - 1)
    def _():
        pltpu.sync_copy(acc_s.at[pl.ds(NS * RPT, RTL)],
                        out_ref.at[c, pl.ds(NS * RPT, RTL)])


def _agg_body(y_ref, srcoff_ref, dst_ref, out_ref,
              sidx_v, didx_v, rows_v, sidx_t, didx_t, rows_t, acc_s, sem):
    c = lax.axis_index("c")
    s = lax.axis_index("s")
    pltpu.sync_copy(y_ref.at[pl.ds(c * N + s * RPT, RPT)],
                    acc_s.at[pl.ds(s * RPT, RPT)])

    @pl.when(s == NS - 1)
    def _():
        pltpu.sync_copy(y_ref.at[pl.ds(c * N + NS * RPT, RTL)],
                        acc_s.at[pl.ds(NS * RPT, RTL)])

    plsc.subcore_barrier()
    base = c * E + s * ET

    @pl.loop(0, NCH)
    def _(k):
        pltpu.sync_copy(srcoff_ref.at[pl.ds(base + k * CH, CH)], sidx_v)
        pltpu.sync_copy(dst_ref.at[pl.ds(base + k * CH, CH)], didx_v)
        pltpu.async_copy(y_ref.at[sidx_v], rows_v, sem).wait()
        pltpu.sync_copy(rows_v, acc_s.at[didx_v], add=True)

    pltpu.sync_copy(srcoff_ref.at[pl.ds(base + NCH * CH, TAIL)], sidx_t)
    pltpu.sync_copy(dst_ref.at[pl.ds(base + NCH * CH, TAIL)], didx_t)
    pltpu.async_copy(y_ref.at[sidx_t], rows_t, sem).wait()
    pltpu.sync_copy(rows_t, acc_s.at[didx_t], add=True)
    plsc.subcore_barrier()
    pltpu.sync_copy(acc_s.at[pl.ds(s * RPT, RPT)],
                    out_ref.at[c, pl.ds(s * RPT, RPT)])

    @pl.when(s == NS - 1)
    def _():
        pltpu.sync_copy(acc_s.at[pl.ds(NS * RPT, RTL)],
                        out_ref.at[c, pl.ds(NS * RPT, RTL)])


def _scale_body(x_ref, w_ref, deg_ref, y_ref):
    xw = jnp.dot(x_ref[...], w_ref[...], preferred_element_type=jnp.float32)
    d0 = lax.rsqrt(deg_ref[0, :, 0:1] + 1.0)
    d1 = lax.rsqrt(deg_ref[1, :, 0:1] + 1.0)
    y_ref[0] = d0 * xw
    y_ref[1] = d1 * xw


_scale_call = pl.pallas_call(
    _scale_body,
    grid=(N // BLK,),
    in_specs=[
        pl.BlockSpec((BLK, D), lambda i: (i, 0)),
        pl.BlockSpec((D, D), lambda i: (0, 0)),
        pl.BlockSpec((2, BLK, D), lambda i: (0, i, 0)),
    ],
    out_specs=pl.BlockSpec((2, BLK, D), lambda i: (0, i, 0)),
    out_shape=jax.ShapeDtypeStruct((2, N, D), jnp.float32),
)


def _final_body(agg_ref, deg_ref, b_ref, out_ref):
    bv = b_ref[0]
    d0 = lax.rsqrt(deg_ref[0, :, 0:1] + 1.0)
    d1 = lax.rsqrt(deg_ref[1, :, 0:1] + 1.0)
    out_ref[:, :D] = d0 * agg_ref[0] + bv
    out_ref[:, D:] = d1 * agg_ref[1] + bv


_final_call = pl.pallas_call(
    _final_body,
    grid=(N // BLK,),
    in_specs=[
        pl.BlockSpec((2, BLK, D), lambda i: (0, i, 0)),
        pl.BlockSpec((2, BLK, D), lambda i: (0, i, 0)),
        pl.BlockSpec((1, D), lambda i: (0, 0)),
    ],
    out_specs=pl.BlockSpec((BLK, 2 * D), lambda i: (i, 0)),
    out_shape=jax.ShapeDtypeStruct((N, 2 * D), jnp.float32),
)


@functools.cache
def _sc_kernels():
    mesh = plsc.VectorSubcoreMesh(core_axis_name="c", subcore_axis_name="s")
    deg_kernel = pl.kernel(
        _deg_body,
        mesh=mesh,
        out_type=jax.ShapeDtypeStruct((2, N, D), jnp.float32),
        scratch_types=[
            pltpu.VMEM((CH,), jnp.int32),
            pltpu.VMEM((TAIL,), jnp.int32),
            pltpu.VMEM((CH, D), jnp.float32),
            pltpu.VMEM_SHARED((N, D), jnp.float32),
            pltpu.SemaphoreType.DMA,
        ],
    )
    agg_kernel = pl.kernel(
        _agg_body,
        mesh=mesh,
        out_type=jax.ShapeDtypeStruct((2, N, D), jnp.float32),
        scratch_types=[
            pltpu.VMEM((CH,), jnp.int32),
            pltpu.VMEM((CH,), jnp.int32),
            pltpu.VMEM((CH, D), jnp.float32),
            pltpu.VMEM((TAIL,), jnp.int32),
            pltpu.VMEM((TAIL,), jnp.int32),
            pltpu.VMEM((TAIL, D), jnp.float32),
            pltpu.VMEM_SHARED((N, D), jnp.float32),
            pltpu.SemaphoreType.DMA,
        ],
    )
    return deg_kernel, agg_kernel


def kernel(x, edge_index_list, W, b):
    deg_kernel, agg_kernel = _sc_kernels()
    ei = edge_index_list.astype(jnp.int32)
    src = ei[:, 0, :]
    dst = ei[:, 1, :].reshape(2 * E)
    srcoff = (src + jnp.arange(2, dtype=jnp.int32)[:, None] * N).reshape(2 * E)
    zerosd = jnp.zeros((N, D), jnp.float32)
    onesd = jnp.ones((CH, D), jnp.float32)

    degp = deg_kernel(dst, zerosd, onesd)
    y = _scale_call(x, W, degp)
    agg = agg_kernel(y.reshape(2 * N, D), srcoff, dst)
    return _final_call(agg, degp, b.reshape(1, D))


# speedup vs baseline: 1.5744x; 1.0997x over previous
import functools

import jax
import jax.numpy as jnp
from jax import lax
from jax.experimental import pallas as pl
from jax.experimental.pallas import tpu as pltpu
from jax.experimental.pallas import tpu_sc as plsc

N = 10000
D = 128
E = 320000
NS = 16
CH = 128
ET = E // NS
NCH = ET // CH
TAIL = ET - NCH * CH
RPT = (N // NS) // 8 * 8
RTL = N - NS * RPT
BLK = 1000


def _deg_body(dst_ref, zeros_ref, ones_ref, out_ref,
              ix0, ix1, ix2, ix3, idx_t, ones_v, acc_s,
              sm0, sm1, sm2, sm3):
    idxs = (ix0, ix1, ix2, ix3)
    sems = (sm0, sm1, sm2, sm3)
    c = lax.axis_index("c")
    s = lax.axis_index("s")
    pltpu.sync_copy(ones_ref, ones_v)
    pltpu.sync_copy(zeros_ref.at[pl.ds(s * RPT, RPT)],
                    acc_s.at[pl.ds(s * RPT, RPT)])

    @pl.when(s == NS - 1)
    def _():
        pltpu.sync_copy(zeros_ref.at[pl.ds(NS * RPT, RTL)],
                        acc_s.at[pl.ds(NS * RPT, RTL)])

    plsc.subcore_barrier()
    base = c * E + s * ET

    def wait_idx(b):
        pltpu.make_async_copy(dst_ref.at[pl.ds(0, CH)], idxs[b],
                              sems[b]).wait()

    for b in range(4):
        pltpu.async_copy(dst_ref.at[pl.ds(base + b * CH, CH)],
                         idxs[b], sems[b])

    @pl.loop(0, NCH // 4 - 1)
    def _(t):
        for b in range(4):
            wait_idx(b)
            pltpu.sync_copy(ones_v, acc_s.at[idxs[b]], add=True)
            pltpu.async_copy(
                dst_ref.at[pl.ds(base + (4 * t + b + 4) * CH, CH)],
                idxs[b], sems[b])

    for b in range(4):
        wait_idx(b)
        pltpu.sync_copy(ones_v, acc_s.at[idxs[b]], add=True)

    pltpu.sync_copy(dst_ref.at[pl.ds(base + NCH * CH, TAIL)], idx_t)
    pltpu.sync_copy(ones_v.at[pl.ds(0, TAIL)], acc_s.at[idx_t], add=True)
    plsc.subcore_barrier()
    pltpu.sync_copy(acc_s.at[pl.ds(s * RPT, RPT)],
                    out_ref.at[c, pl.ds(s * RPT, RPT)])

    @pl.when(s == NS - 1)
    def _():
        pltpu.sync_copy(acc_s.at[pl.ds(NS * RPT, RTL)],
                        out_ref.at[c, pl.ds(NS * RPT, RTL)])


def _agg_body(y_ref, srcoff_ref, dst_ref, out_ref,
              sidx_v, didx_v, rows_v, sidx_t, didx_t, rows_t, acc_s, sem):
    c = lax.axis_index("c")
    s = lax.axis_index("s")
    pltpu.sync_copy(y_ref.at[pl.ds(c * N + s * RPT, RPT)],
                    acc_s.at[pl.ds(s * RPT, RPT)])

    @pl.when(s == NS - 1)
    def _():
        pltpu.sync_copy(y_ref.at[pl.ds(c * N + NS * RPT, RTL)],
                        acc_s.at[pl.ds(NS * RPT, RTL)])

    plsc.subcore_barrier()
    base = c * E + s * ET

    @pl.loop(0, NCH)
    def _(k):
        pltpu.sync_copy(srcoff_ref.at[pl.ds(base + k * CH, CH)], sidx_v)
        pltpu.sync_copy(dst_ref.at[pl.ds(base + k * CH, CH)], didx_v)
        pltpu.async_copy(y_ref.at[sidx_v], rows_v, sem).wait()
        pltpu.sync_copy(rows_v, acc_s.at[didx_v], add=True)

    pltpu.sync_copy(srcoff_ref.at[pl.ds(base + NCH * CH, TAIL)], sidx_t)
    pltpu.sync_copy(dst_ref.at[pl.ds(base + NCH * CH, TAIL)], didx_t)
    pltpu.async_copy(y_ref.at[sidx_t], rows_t, sem).wait()
    pltpu.sync_copy(rows_t, acc_s.at[didx_t], add=True)
    plsc.subcore_barrier()
    pltpu.sync_copy(acc_s.at[pl.ds(s * RPT, RPT)],
                    out_ref.at[c, pl.ds(s * RPT, RPT)])

    @pl.when(s == NS - 1)
    def _():
        pltpu.sync_copy(acc_s.at[pl.ds(NS * RPT, RTL)],
                        out_ref.at[c, pl.ds(NS * RPT, RTL)])


def _scale_body(x_ref, w_ref, deg_ref, y_ref):
    xw = jnp.dot(x_ref[...], w_ref[...], preferred_element_type=jnp.float32)
    d0 = lax.rsqrt(deg_ref[0, :, 0:1] + 1.0)
    d1 = lax.rsqrt(deg_ref[1, :, 0:1] + 1.0)
    y_ref[0] = d0 * xw
    y_ref[1] = d1 * xw


_scale_call = pl.pallas_call(
    _scale_body,
    grid=(N // BLK,),
    in_specs=[
        pl.BlockSpec((BLK, D), lambda i: (i, 0)),
        pl.BlockSpec((D, D), lambda i: (0, 0)),
        pl.BlockSpec((2, BLK, D), lambda i: (0, i, 0)),
    ],
    out_specs=pl.BlockSpec((2, BLK, D), lambda i: (0, i, 0)),
    out_shape=jax.ShapeDtypeStruct((2, N, D), jnp.float32),
)


def _final_body(agg_ref, deg_ref, b_ref, out_ref):
    bv = b_ref[0]
    d0 = lax.rsqrt(deg_ref[0, :, 0:1] + 1.0)
    d1 = lax.rsqrt(deg_ref[1, :, 0:1] + 1.0)
    out_ref[:, :D] = d0 * agg_ref[0] + bv
    out_ref[:, D:] = d1 * agg_ref[1] + bv


_final_call = pl.pallas_call(
    _final_body,
    grid=(N // BLK,),
    in_specs=[
        pl.BlockSpec((2, BLK, D), lambda i: (0, i, 0)),
        pl.BlockSpec((2, BLK, D), lambda i: (0, i, 0)),
        pl.BlockSpec((1, D), lambda i: (0, 0)),
    ],
    out_specs=pl.BlockSpec((BLK, 2 * D), lambda i: (i, 0)),
    out_shape=jax.ShapeDtypeStruct((N, 2 * D), jnp.float32),
)


@functools.cache
def _sc_kernels():
    mesh = plsc.VectorSubcoreMesh(core_axis_name="c", subcore_axis_name="s")
    deg_kernel = pl.kernel(
        _deg_body,
        mesh=mesh,
        out_type=jax.ShapeDtypeStruct((2, N, D), jnp.float32),
        scratch_types=[
            pltpu.VMEM((CH,), jnp.int32),
            pltpu.VMEM((CH,), jnp.int32),
            pltpu.VMEM((CH,), jnp.int32),
            pltpu.VMEM((CH,), jnp.int32),
            pltpu.VMEM((TAIL,), jnp.int32),
            pltpu.VMEM((CH, D), jnp.float32),
            pltpu.VMEM_SHARED((N, D), jnp.float32),
            pltpu.SemaphoreType.DMA,
            pltpu.SemaphoreType.DMA,
            pltpu.SemaphoreType.DMA,
            pltpu.SemaphoreType.DMA,
        ],
    )
    agg_kernel = pl.kernel(
        _agg_body,
        mesh=mesh,
        out_type=jax.ShapeDtypeStruct((2, N, D), jnp.float32),
        scratch_types=[
            pltpu.VMEM((CH,), jnp.int32),
            pltpu.VMEM((CH,), jnp.int32),
            pltpu.VMEM((CH, D), jnp.float32),
            pltpu.VMEM((TAIL,), jnp.int32),
            pltpu.VMEM((TAIL,), jnp.int32),
            pltpu.VMEM((TAIL, D), jnp.float32),
            pltpu.VMEM_SHARED((N, D), jnp.float32),
            pltpu.SemaphoreType.DMA,
        ],
    )
    return deg_kernel, agg_kernel


def kernel(x, edge_index_list, W, b):
    deg_kernel, agg_kernel = _sc_kernels()
    ei = edge_index_list.astype(jnp.int32)
    src = ei[:, 0, :]
    dst = ei[:, 1, :].reshape(2 * E)
    srcoff = (src + jnp.arange(2, dtype=jnp.int32)[:, None] * N).reshape(2 * E)
    zerosd = jnp.zeros((N, D), jnp.float32)
    onesd = jnp.ones((CH, D), jnp.float32)

    degp = deg_kernel(dst, zerosd, onesd)
    y = _scale_call(x, W, degp)
    agg = agg_kernel(y.reshape(2 * N, D), srcoff, dst)
    return _final_call(agg, degp, b.reshape(1, D))


# speedup vs baseline: 1.9898x; 1.2638x over previous
import functools

import jax
import jax.numpy as jnp
from jax import lax
from jax.experimental import pallas as pl
from jax.experimental.pallas import tpu as pltpu
from jax.experimental.pallas import tpu_sc as plsc

N = 10000
D = 128
E = 320000
NS = 16
CH = 128
ET = E // NS
NCH = ET // CH
TAIL = ET - NCH * CH
RPT = (N // NS) // 8 * 8
RTL = N - NS * RPT
BLK = 1000


def _deg_body(dst_ref, zeros_ref, ones_ref, out_ref,
              ix0, ix1, ix2, ix3, idx_t, ones_v, acc_s,
              sm0, sm1, sm2, sm3):
    idxs = (ix0, ix1, ix2, ix3)
    sems = (sm0, sm1, sm2, sm3)
    c = lax.axis_index("c")
    s = lax.axis_index("s")
    pltpu.sync_copy(ones_ref, ones_v)
    pltpu.sync_copy(zeros_ref.at[pl.ds(s * RPT, RPT)],
                    acc_s.at[pl.ds(s * RPT, RPT)])

    @pl.when(s == NS - 1)
    def _():
        pltpu.sync_copy(zeros_ref.at[pl.ds(NS * RPT, RTL)],
                        acc_s.at[pl.ds(NS * RPT, RTL)])

    plsc.subcore_barrier()
    base = c * E + s * ET

    def wait_idx(b):
        pltpu.make_async_copy(dst_ref.at[pl.ds(0, CH)], idxs[b],
                              sems[b]).wait()

    for b in range(4):
        pltpu.async_copy(dst_ref.at[pl.ds(base + b * CH, CH)],
                         idxs[b], sems[b])

    @pl.loop(0, NCH // 4 - 1)
    def _(t):
        for b in range(4):
            wait_idx(b)
            pltpu.sync_copy(ones_v, acc_s.at[idxs[b]], add=True)
            pltpu.async_copy(
                dst_ref.at[pl.ds(base + (4 * t + b + 4) * CH, CH)],
                idxs[b], sems[b])

    for b in range(4):
        wait_idx(b)
        pltpu.sync_copy(ones_v, acc_s.at[idxs[b]], add=True)

    pltpu.sync_copy(dst_ref.at[pl.ds(base + NCH * CH, TAIL)], idx_t)
    pltpu.sync_copy(ones_v.at[pl.ds(0, TAIL)], acc_s.at[idx_t], add=True)
    plsc.subcore_barrier()
    pltpu.sync_copy(acc_s.at[pl.ds(s * RPT, RPT)],
                    out_ref.at[c, pl.ds(s * RPT, RPT)])

    @pl.when(s == NS - 1)
    def _():
        pltpu.sync_copy(acc_s.at[pl.ds(NS * RPT, RTL)],
                        out_ref.at[c, pl.ds(NS * RPT, RTL)])


def _agg_body(y_ref, srcoff_ref, dst_ref, out_ref,
              sx0, sx1, sx2, sx3, dx0, dx1, dx2, dx3, rows_v,
              sidx_t, didx_t, rows_t, acc_s,
              im0, im1, im2, im3, sem):
    sidxs = (sx0, sx1, sx2, sx3)
    didxs = (dx0, dx1, dx2, dx3)
    isems = (im0, im1, im2, im3)
    c = lax.axis_index("c")
    s = lax.axis_index("s")
    pltpu.sync_copy(y_ref.at[pl.ds(c * N + s * RPT, RPT)],
                    acc_s.at[pl.ds(s * RPT, RPT)])

    @pl.when(s == NS - 1)
    def _():
        pltpu.sync_copy(y_ref.at[pl.ds(c * N + NS * RPT, RTL)],
                        acc_s.at[pl.ds(NS * RPT, RTL)])

    plsc.subcore_barrier()
    base = c * E + s * ET

    def start_idx(b, k):
        pltpu.async_copy(srcoff_ref.at[pl.ds(base + k * CH, CH)],
                         sidxs[b], isems[b])
        pltpu.async_copy(dst_ref.at[pl.ds(base + k * CH, CH)],
                         didxs[b], isems[b])

    def wait_idx(b):
        pltpu.make_async_copy(srcoff_ref.at[pl.ds(0, CH)], sidxs[b],
                              isems[b]).wait()
        pltpu.make_async_copy(srcoff_ref.at[pl.ds(0, CH)], didxs[b],
                              isems[b]).wait()

    for b in range(4):
        start_idx(b, b)

    @pl.loop(0, NCH // 4 - 1)
    def _(t):
        for b in range(4):
            wait_idx(b)
            pltpu.async_copy(y_ref.at[sidxs[b]], rows_v, sem).wait()
            pltpu.sync_copy(rows_v, acc_s.at[didxs[b]], add=True)
            start_idx(b, 4 * t + b + 4)

    for b in range(4):
        wait_idx(b)
        pltpu.async_copy(y_ref.at[sidxs[b]], rows_v, sem).wait()
        pltpu.sync_copy(rows_v, acc_s.at[didxs[b]], add=True)

    pltpu.sync_copy(srcoff_ref.at[pl.ds(base + NCH * CH, TAIL)], sidx_t)
    pltpu.sync_copy(dst_ref.at[pl.ds(base + NCH * CH, TAIL)], didx_t)
    pltpu.async_copy(y_ref.at[sidx_t], rows_t, sem).wait()
    pltpu.sync_copy(rows_t, acc_s.at[didx_t], add=True)
    plsc.subcore_barrier()
    pltpu.sync_copy(acc_s.at[pl.ds(s * RPT, RPT)],
                    out_ref.at[c, pl.ds(s * RPT, RPT)])

    @pl.when(s == NS - 1)
    def _():
        pltpu.sync_copy(acc_s.at[pl.ds(NS * RPT, RTL)],
                        out_ref.at[c, pl.ds(NS * RPT, RTL)])


def _scale_body(x_ref, w_ref, deg_ref, y_ref):
    xw = jnp.dot(x_ref[...], w_ref[...], preferred_element_type=jnp.float32)
    d0 = lax.rsqrt(deg_ref[0, :, 0:1] + 1.0)
    d1 = lax.rsqrt(deg_ref[1, :, 0:1] + 1.0)
    y_ref[0] = d0 * xw
    y_ref[1] = d1 * xw


_scale_call = pl.pallas_call(
    _scale_body,
    grid=(N // BLK,),
    in_specs=[
        pl.BlockSpec((BLK, D), lambda i: (i, 0)),
        pl.BlockSpec((D, D), lambda i: (0, 0)),
        pl.BlockSpec((2, BLK, D), lambda i: (0, i, 0)),
    ],
    out_specs=pl.BlockSpec((2, BLK, D), lambda i: (0, i, 0)),
    out_shape=jax.ShapeDtypeStruct((2, N, D), jnp.float32),
)


def _final_body(agg_ref, deg_ref, b_ref, out_ref):
    bv = b_ref[0]
    d0 = lax.rsqrt(deg_ref[0, :, 0:1] + 1.0)
    d1 = lax.rsqrt(deg_ref[1, :, 0:1] + 1.0)
    out_ref[:, :D] = d0 * agg_ref[0] + bv
    out_ref[:, D:] = d1 * agg_ref[1] + bv


_final_call = pl.pallas_call(
    _final_body,
    grid=(N // BLK,),
    in_specs=[
        pl.BlockSpec((2, BLK, D), lambda i: (0, i, 0)),
        pl.BlockSpec((2, BLK, D), lambda i: (0, i, 0)),
        pl.BlockSpec((1, D), lambda i: (0, 0)),
    ],
    out_specs=pl.BlockSpec((BLK, 2 * D), lambda i: (i, 0)),
    out_shape=jax.ShapeDtypeStruct((N, 2 * D), jnp.float32),
)


@functools.cache
def _sc_kernels():
    mesh = plsc.VectorSubcoreMesh(core_axis_name="c", subcore_axis_name="s")
    deg_kernel = pl.kernel(
        _deg_body,
        mesh=mesh,
        out_type=jax.ShapeDtypeStruct((2, N, D), jnp.float32),
        scratch_types=[
            pltpu.VMEM((CH,), jnp.int32),
            pltpu.VMEM((CH,), jnp.int32),
            pltpu.VMEM((CH,), jnp.int32),
            pltpu.VMEM((CH,), jnp.int32),
            pltpu.VMEM((TAIL,), jnp.int32),
            pltpu.VMEM((CH, D), jnp.float32),
            pltpu.VMEM_SHARED((N, D), jnp.float32),
            pltpu.SemaphoreType.DMA,
            pltpu.SemaphoreType.DMA,
            pltpu.SemaphoreType.DMA,
            pltpu.SemaphoreType.DMA,
        ],
    )
    agg_kernel = pl.kernel(
        _agg_body,
        mesh=mesh,
        out_type=jax.ShapeDtypeStruct((2, N, D), jnp.float32),
        scratch_types=[
            pltpu.VMEM((CH,), jnp.int32),
            pltpu.VMEM((CH,), jnp.int32),
            pltpu.VMEM((CH,), jnp.int32),
            pltpu.VMEM((CH,), jnp.int32),
            pltpu.VMEM((CH,), jnp.int32),
            pltpu.VMEM((CH,), jnp.int32),
            pltpu.VMEM((CH,), jnp.int32),
            pltpu.VMEM((CH,), jnp.int32),
            pltpu.VMEM((CH, D), jnp.float32),
            pltpu.VMEM((TAIL,), jnp.int32),
            pltpu.VMEM((TAIL,), jnp.int32),
            pltpu.VMEM((TAIL, D), jnp.float32),
            pltpu.VMEM_SHARED((N, D), jnp.float32),
            pltpu.SemaphoreType.DMA,
            pltpu.SemaphoreType.DMA,
            pltpu.SemaphoreType.DMA,
            pltpu.SemaphoreType.DMA,
            pltpu.SemaphoreType.DMA,
        ],
    )
    return deg_kernel, agg_kernel


def kernel(x, edge_index_list, W, b):
    deg_kernel, agg_kernel = _sc_kernels()
    ei = edge_index_list.astype(jnp.int32)
    src = ei[:, 0, :]
    dst = ei[:, 1, :].reshape(2 * E)
    srcoff = (src + jnp.arange(2, dtype=jnp.int32)[:, None] * N).reshape(2 * E)
    zerosd = jnp.zeros((N, D), jnp.float32)
    onesd = jnp.ones((CH, D), jnp.float32)

    degp = deg_kernel(dst, zerosd, onesd)
    y = _scale_call(x, W, degp)
    agg = agg_kernel(y.reshape(2 * N, D), srcoff, dst)
    return _final_call(agg, degp, b.reshape(1, D))


# speedup vs baseline: 2.6987x; 1.3563x over previous
import functools

import jax
import jax.numpy as jnp
from jax import lax
from jax.experimental import pallas as pl
from jax.experimental.pallas import tpu as pltpu
from jax.experimental.pallas import tpu_sc as plsc

N = 10000
D = 128
E = 320000
NS = 16
CH = 128
ET = E // NS
NCH = ET // CH
TAIL = ET - NCH * CH
RPT = (N // NS) // 8 * 8
RTL = N - NS * RPT
BLK = 1000


def _deg_body(dst_ref, zeros_ref, ones_ref, out_ref,
              ix0, ix1, ix2, ix3, idx_t, ones_v, acc_s,
              sm0, sm1, sm2, sm3):
    idxs = (ix0, ix1, ix2, ix3)
    sems = (sm0, sm1, sm2, sm3)
    c = lax.axis_index("c")
    s = lax.axis_index("s")
    pltpu.sync_copy(ones_ref, ones_v)
    pltpu.sync_copy(zeros_ref.at[pl.ds(s * RPT, RPT)],
                    acc_s.at[pl.ds(s * RPT, RPT)])

    @pl.when(s == NS - 1)
    def _():
        pltpu.sync_copy(zeros_ref.at[pl.ds(NS * RPT, RTL)],
                        acc_s.at[pl.ds(NS * RPT, RTL)])

    plsc.subcore_barrier()
    base = c * E + s * ET

    def wait_idx(b):
        pltpu.make_async_copy(dst_ref.at[pl.ds(0, CH)], idxs[b],
                              sems[b]).wait()

    for b in range(4):
        pltpu.async_copy(dst_ref.at[pl.ds(base + b * CH, CH)],
                         idxs[b], sems[b])

    @pl.loop(0, NCH // 4 - 1)
    def _(t):
        for b in range(4):
            wait_idx(b)
            pltpu.sync_copy(ones_v, acc_s.at[idxs[b]], add=True)
            pltpu.async_copy(
                dst_ref.at[pl.ds(base + (4 * t + b + 4) * CH, CH)],
                idxs[b], sems[b])

    for b in range(4):
        wait_idx(b)
        pltpu.sync_copy(ones_v, acc_s.at[idxs[b]], add=True)

    pltpu.sync_copy(dst_ref.at[pl.ds(base + NCH * CH, TAIL)], idx_t)
    pltpu.sync_copy(ones_v.at[pl.ds(0, TAIL)], acc_s.at[idx_t], add=True)
    plsc.subcore_barrier()
    pltpu.sync_copy(acc_s.at[pl.ds(s * RPT, RPT)],
                    out_ref.at[c, pl.ds(s * RPT, RPT)])

    @pl.when(s == NS - 1)
    def _():
        pltpu.sync_copy(acc_s.at[pl.ds(NS * RPT, RTL)],
                        out_ref.at[c, pl.ds(NS * RPT, RTL)])


def _agg_body(y_ref, srcoff_ref, dst_ref, out_ref,
              sx0, sx1, sx2, sx3, dx0, dx1, dx2, dx3, rows_v, rows_w,
              sidx_t, didx_t, rows_t, acc_s,
              im0, im1, im2, im3, sem, semw):
    sidxs = (sx0, sx1, sx2, sx3)
    didxs = (dx0, dx1, dx2, dx3)
    isems = (im0, im1, im2, im3)
    c = lax.axis_index("c")
    s = lax.axis_index("s")
    pltpu.sync_copy(y_ref.at[pl.ds(c * N + s * RPT, RPT)],
                    acc_s.at[pl.ds(s * RPT, RPT)])

    @pl.when(s == NS - 1)
    def _():
        pltpu.sync_copy(y_ref.at[pl.ds(c * N + NS * RPT, RTL)],
                        acc_s.at[pl.ds(NS * RPT, RTL)])

    plsc.subcore_barrier()
    base = c * E + s * ET

    def start_idx(b, k):
        pltpu.async_copy(srcoff_ref.at[pl.ds(base + k * CH, CH)],
                         sidxs[b], isems[b])
        pltpu.async_copy(dst_ref.at[pl.ds(base + k * CH, CH)],
                         didxs[b], isems[b])

    def wait_idx(b):
        pltpu.make_async_copy(srcoff_ref.at[pl.ds(0, CH)], sidxs[b],
                              isems[b]).wait()
        pltpu.make_async_copy(srcoff_ref.at[pl.ds(0, CH)], didxs[b],
                              isems[b]).wait()

    rows = (rows_v, rows_w)
    gsems = (sem, semw)

    def wait_gather(rb):
        pltpu.make_async_copy(y_ref.at[pl.ds(0, CH)], rows[rb],
                              gsems[rb]).wait()

    for b in range(4):
        start_idx(b, b)
    wait_idx(0)
    pltpu.async_copy(y_ref.at[sidxs[0]], rows[0], gsems[0])

    @pl.loop(0, NCH // 4 - 1)
    def _(t):
        for b in range(4):
            bn = (b + 1) % 4
            wait_idx(bn)
            pltpu.async_copy(y_ref.at[sidxs[bn]], rows[bn % 2],
                             gsems[bn % 2])
            wait_gather(b % 2)
            pltpu.sync_copy(rows[b % 2], acc_s.at[didxs[b]], add=True)
            start_idx(b, 4 * t + b + 4)

    for b in range(3):
        bn = b + 1
        wait_idx(bn)
        pltpu.async_copy(y_ref.at[sidxs[bn]], rows[bn % 2], gsems[bn % 2])
        wait_gather(b % 2)
        pltpu.sync_copy(rows[b % 2], acc_s.at[didxs[b]], add=True)
    wait_gather(3 % 2)
    pltpu.sync_copy(rows[3 % 2], acc_s.at[didxs[3]], add=True)

    pltpu.sync_copy(srcoff_ref.at[pl.ds(base + NCH * CH, TAIL)], sidx_t)
    pltpu.sync_copy(dst_ref.at[pl.ds(base + NCH * CH, TAIL)], didx_t)
    pltpu.async_copy(y_ref.at[sidx_t], rows_t, sem).wait()
    pltpu.sync_copy(rows_t, acc_s.at[didx_t], add=True)
    plsc.subcore_barrier()
    pltpu.sync_copy(acc_s.at[pl.ds(s * RPT, RPT)],
                    out_ref.at[c, pl.ds(s * RPT, RPT)])

    @pl.when(s == NS - 1)
    def _():
        pltpu.sync_copy(acc_s.at[pl.ds(NS * RPT, RTL)],
                        out_ref.at[c, pl.ds(NS * RPT, RTL)])


def _scale_body(x_ref, w_ref, deg_ref, y_ref):
    xw = jnp.dot(x_ref[...], w_ref[...], preferred_element_type=jnp.float32)
    d0 = lax.rsqrt(deg_ref[0, :, 0:1] + 1.0)
    d1 = lax.rsqrt(deg_ref[1, :, 0:1] + 1.0)
    y_ref[0] = d0 * xw
    y_ref[1] = d1 * xw


_scale_call = pl.pallas_call(
    _scale_body,
    grid=(N // BLK,),
    in_specs=[
        pl.BlockSpec((BLK, D), lambda i: (i, 0)),
        pl.BlockSpec((D, D), lambda i: (0, 0)),
        pl.BlockSpec((2, BLK, D), lambda i: (0, i, 0)),
    ],
    out_specs=pl.BlockSpec((2, BLK, D), lambda i: (0, i, 0)),
    out_shape=jax.ShapeDtypeStruct((2, N, D), jnp.float32),
)


def _final_body(agg_ref, deg_ref, b_ref, out_ref):
    bv = b_ref[0]
    d0 = lax.rsqrt(deg_ref[0, :, 0:1] + 1.0)
    d1 = lax.rsqrt(deg_ref[1, :, 0:1] + 1.0)
    out_ref[:, :D] = d0 * agg_ref[0] + bv
    out_ref[:, D:] = d1 * agg_ref[1] + bv


_final_call = pl.pallas_call(
    _final_body,
    grid=(N // BLK,),
    in_specs=[
        pl.BlockSpec((2, BLK, D), lambda i: (0, i, 0)),
        pl.BlockSpec((2, BLK, D), lambda i: (0, i, 0)),
        pl.BlockSpec((1, D), lambda i: (0, 0)),
    ],
    out_specs=pl.BlockSpec((BLK, 2 * D), lambda i: (i, 0)),
    out_shape=jax.ShapeDtypeStruct((N, 2 * D), jnp.float32),
)


@functools.cache
def _sc_kernels():
    mesh = plsc.VectorSubcoreMesh(core_axis_name="c", subcore_axis_name="s")
    deg_kernel = pl.kernel(
        _deg_body,
        mesh=mesh,
        out_type=jax.ShapeDtypeStruct((2, N, D), jnp.float32),
        scratch_types=[
            pltpu.VMEM((CH,), jnp.int32),
            pltpu.VMEM((CH,), jnp.int32),
            pltpu.VMEM((CH,), jnp.int32),
            pltpu.VMEM((CH,), jnp.int32),
            pltpu.VMEM((TAIL,), jnp.int32),
            pltpu.VMEM((CH, D), jnp.float32),
            pltpu.VMEM_SHARED((N, D), jnp.float32),
            pltpu.SemaphoreType.DMA,
            pltpu.SemaphoreType.DMA,
            pltpu.SemaphoreType.DMA,
            pltpu.SemaphoreType.DMA,
        ],
    )
    agg_kernel = pl.kernel(
        _agg_body,
        mesh=mesh,
        out_type=jax.ShapeDtypeStruct((2, N, D), jnp.float32),
        scratch_types=[
            pltpu.VMEM((CH,), jnp.int32),
            pltpu.VMEM((CH,), jnp.int32),
            pltpu.VMEM((CH,), jnp.int32),
            pltpu.VMEM((CH,), jnp.int32),
            pltpu.VMEM((CH,), jnp.int32),
            pltpu.VMEM((CH,), jnp.int32),
            pltpu.VMEM((CH,), jnp.int32),
            pltpu.VMEM((CH,), jnp.int32),
            pltpu.VMEM((CH, D), jnp.float32),
            pltpu.VMEM((CH, D), jnp.float32),
            pltpu.VMEM((TAIL,), jnp.int32),
            pltpu.VMEM((TAIL,), jnp.int32),
            pltpu.VMEM((TAIL, D), jnp.float32),
            pltpu.VMEM_SHARED((N, D), jnp.float32),
            pltpu.SemaphoreType.DMA,
            pltpu.SemaphoreType.DMA,
            pltpu.SemaphoreType.DMA,
            pltpu.SemaphoreType.DMA,
            pltpu.SemaphoreType.DMA,
            pltpu.SemaphoreType.DMA,
        ],
    )
    return deg_kernel, agg_kernel


def kernel(x, edge_index_list, W, b):
    deg_kernel, agg_kernel = _sc_kernels()
    ei = edge_index_list.astype(jnp.int32)
    src = ei[:, 0, :]
    dst = ei[:, 1, :].reshape(2 * E)
    srcoff = (src + jnp.arange(2, dtype=jnp.int32)[:, None] * N).reshape(2 * E)
    zerosd = jnp.zeros((N, D), jnp.float32)
    onesd = jnp.ones((CH, D), jnp.float32)

    degp = deg_kernel(dst, zerosd, onesd)
    y = _scale_call(x, W, degp)
    agg = agg_kernel(y.reshape(2 * N, D), srcoff, dst)
    return _final_call(agg, degp, b.reshape(1, D))
